# baseline (speedup 1.0000x reference)
import functools

import jax
import jax.numpy as jnp
from jax import lax
from jax.experimental import pallas as pl
from jax.experimental.pallas import tpu as pltpu
from jax.experimental.pallas import tpu_sc as plsc

B = 16384
NF = 8
D = 64
HID = NF * D
VOCAB = 1000000
NITEMS = 1000000

NW = 32
RPW = B // NW
CH = 128
NCH = RPW // CH


@functools.cache
def _mesh():
    return plsc.VectorSubcoreMesh(core_axis_name="c", subcore_axis_name="s")


@functools.cache
def _sc_gather():
    @functools.partial(
        pl.kernel,
        mesh=_mesh(),
        out_type=[
            jax.ShapeDtypeStruct((NF, B, 128), jnp.float32),
            jax.ShapeDtypeStruct((NF, B, 128), jnp.float32),
            jax.ShapeDtypeStruct((NF, B), jnp.int32),
        ],
        scratch_types=[
            pltpu.VMEM((NF, RPW), jnp.int32),
            pltpu.VMEM((RPW,), jnp.int32),
            pltpu.VMEM((RPW,), jnp.int32),
            pltpu.VMEM((CH, 16 * NF), jnp.int32),
            pltpu.VMEM((NF, RPW), jnp.int32),
            pltpu.VMEM((CH,), jnp.int32),
            pltpu.VMEM((2, CH, 128), jnp.float32),
            pltpu.SemaphoreType.DMA,
            pltpu.SemaphoreType.DMA,
        ],
        compiler_params=pltpu.CompilerParams(needs_layout_passes=False),
    )
    def body(ufT, item_ids, fld, tab2, out_u, out_i, out_ifi,
             uidx, iid, irow, frows, ifi, qv, pbuf, gsem, wsem):
        w = lax.axis_index("s") * 2 + lax.axis_index("c")
        b0 = w * RPW

        pltpu.sync_copy(ufT.at[:, pl.ds(b0, RPW)], uidx)
        pltpu.sync_copy(item_ids.at[pl.ds(b0, RPW)], iid)

        for k in range(RPW // 16):
            sl = pl.ds(16 * k, 16)
            irow[sl] = iid[sl] >> 4

        for c in range(NCH):
            pltpu.async_copy(fld.at[irow.at[pl.ds(c * CH, CH)]], frows,
                             gsem).wait()
            for k in range(CH // 16):
                j = c * CH + 16 * k
                rows16 = lax.iota(jnp.int32, 16) + 16 * k
                off16 = (iid[pl.ds(j, 16)] & 15) * NF
                for f in range(NF):
                    ifi[f, pl.ds(j, 16)] = plsc.load_gather(
                        frows, [rows16, off16 + f])
        pltpu.sync_copy(ifi, out_ifi.at[:, pl.ds(b0, RPW)])

        for src, out in ((uidx, out_u), (ifi, out_i)):
            for f in range(NF):
                for c in range(NCH):
                    for k in range(CH // 16):
                        sl = pl.ds(16 * k, 16)
                        qv[sl] = src[f, pl.ds(c * CH + 16 * k, 16)] >> 1
                    buf = pbuf.at[c % 2]
                    pltpu.async_copy(tab2.at[qv], buf, gsem).wait()
                    pltpu.sync_copy(
                        buf, out.at[f, pl.ds(b0 + c * CH, CH), :])

    return body


BLK = 1024
NB = B // BLK


def _mlp_body(up, ip, ufT, ifi, uW1, ub1, uW2, ub2, uW3, ub3,
              iW1, ib1, iW2, ib2, out):
    f32 = jnp.float32

    def dot(a, b):
        return jnp.dot(a, b, preferred_element_type=f32)

    def tower_l1(pairs, fids, W1, b1):
        acc = b1[...]
        for f in range(NF):
            par = (fids[f, :] & 1)[:, None]
            pf = pairs[f]
            ef = jnp.where(par == 1, pf[:, D:], pf[:, :D])
            acc = acc + dot(ef, W1[pl.ds(D * f, D), :])
        return jnp.maximum(acc, 0.0)

    h = tower_l1(up, ufT, uW1, ub1)
    g = tower_l1(ip, ifi, iW1, ib1)
    h = jnp.maximum(dot(h, uW2[...]) + ub2[...], 0.0)
    uv = dot(h, uW3[...]) + ub3[...]
    iv = dot(g, iW2[...]) + ib2[...]
    out[...] = jnp.sum(uv * iv, axis=1)


def _full(shape):
    return pl.BlockSpec(shape, lambda i: tuple(0 for _ in shape))


_mlp_call = pl.pallas_call(
    _mlp_body,
    grid=(NB,),
    in_specs=[
        pl.BlockSpec((NF, BLK, 128), lambda i: (0, i, 0)),
        pl.BlockSpec((NF, BLK, 128), lambda i: (0, i, 0)),
        pl.BlockSpec((NF, BLK), lambda i: (0, i)),
        pl.BlockSpec((NF, BLK), lambda i: (0, i)),
        _full((HID, HID // 2)),
        _full((1, HID // 2)),
        _full((HID // 2, HID // 4)),
        _full((1, HID // 4)),
        _full((HID // 4, D)),
        _full((1, D)),
        _full((HID, HID // 2)),
        _full((1, HID // 2)),
        _full((HID // 2, D)),
        _full((1, D)),
    ],
    out_specs=pl.BlockSpec((BLK,), lambda i: (i,)),
    out_shape=jax.ShapeDtypeStruct((B,), jnp.float32),
)


def kernel(user_feats, item_ids, item_feats_table, embed_table,
           uW1, ub1, uW2, ub2, uW3, ub3, iW1, ib1, iW2, ib2):
    tab2 = embed_table.reshape(VOCAB // 2, 128)
    fld = item_feats_table.reshape(NITEMS // 16, 128)
    up, ip, ifi = _sc_gather()(user_feats.T, item_ids, fld, tab2)
    scores = _mlp_call(up, ip, user_feats.T, ifi,
                       uW1, ub1.reshape(1, -1), uW2, ub2.reshape(1, -1),
                       uW3, ub3.reshape(1, -1), iW1, ib1.reshape(1, -1),
                       iW2, ib2.reshape(1, -1))
    return scores


# speedup vs baseline: 1.1164x; 1.1164x over previous
import functools

import jax
import jax.numpy as jnp
from jax import lax
from jax.experimental import pallas as pl
from jax.experimental.pallas import tpu as pltpu
from jax.experimental.pallas import tpu_sc as plsc

B = 16384
NF = 8
D = 64
HID = NF * D
VOCAB = 1000000
NITEMS = 1000000

NW = 32
RPW = B // NW
CH = 128
NCH = RPW // CH
NG = HID // 128


@functools.cache
def _mesh():
    return plsc.VectorSubcoreMesh(core_axis_name="c", subcore_axis_name="s")


@functools.cache
def _sc_gather():
    @functools.partial(
        pl.kernel,
        mesh=_mesh(),
        out_type=[
            jax.ShapeDtypeStruct((NG, B, 128), jnp.float32),
            jax.ShapeDtypeStruct((NG, B, 128), jnp.float32),
        ],
        scratch_types=[
            pltpu.VMEM((NF, RPW), jnp.int32),
            pltpu.VMEM((RPW,), jnp.int32),
            pltpu.VMEM((CH, NF), jnp.int32),
            pltpu.VMEM((NF, RPW), jnp.int32),
            pltpu.VMEM((2, CH, D), jnp.float32),
            pltpu.SemaphoreType.DMA,
            pltpu.SemaphoreType.DMA,
        ],
        compiler_params=pltpu.CompilerParams(use_tc_tiling_on_sc=False,
                                             needs_layout_passes=False),
    )
    def body(ufT, item_ids, ift, tab, out_u, out_i,
             uidx, iid, frows, ifi, ebuf, gsem, wsem):
        w = lax.axis_index("s") * 2 + lax.axis_index("c")
        b0 = w * RPW

        pltpu.sync_copy(ufT.at[:, pl.ds(b0, RPW)], uidx)
        pltpu.sync_copy(item_ids.at[pl.ds(b0, RPW)], iid)

        for c in range(NCH):
            pltpu.async_copy(ift.at[iid.at[pl.ds(c * CH, CH)]], frows,
                             gsem).wait()
            for k in range(CH // 16):
                rows16 = lax.iota(jnp.int32, 16) + 16 * k
                for f in range(NF):
                    ifi[f, pl.ds(c * CH + 16 * k, 16)] = plsc.load_gather(
                        frows, [rows16, jnp.full((16,), f, jnp.int32)])

        for src, out in ((uidx, out_u), (ifi, out_i)):
            for f in range(NF):
                g, h = f // 2, (f % 2) * D
                for c in range(NCH):
                    buf = ebuf.at[c % 2]
                    idx = src.at[f].at[pl.ds(c * CH, CH)]
                    pltpu.async_copy(tab.at[idx], buf, gsem).wait()
                    pltpu.sync_copy(
                        buf, out.at[g, pl.ds(b0 + c * CH, CH), pl.ds(h, D)])

    return body


BLK = 1024
NB = B // BLK


def _mlp_body(ue, ie, uW1, ub1, uW2, ub2, uW3, ub3, iW1, ib1, iW2, ib2, out):
    f32 = jnp.float32

    def dot(a, b):
        return jnp.dot(a, b, preferred_element_type=f32)

    h = ub1[...]
    g = ib1[...]
    for p in range(NG):
        h = h + dot(ue[p], uW1[pl.ds(128 * p, 128), :])
        g = g + dot(ie[p], iW1[pl.ds(128 * p, 128), :])
    h = jnp.maximum(h, 0.0)
    g = jnp.maximum(g, 0.0)
    h = jnp.maximum(dot(h, uW2[...]) + ub2[...], 0.0)
    uv = dot(h, uW3[...]) + ub3[...]
    iv = dot(g, iW2[...]) + ib2[...]
    out[...] = jnp.sum(uv * iv, axis=1)


def _full(shape):
    return pl.BlockSpec(shape, lambda i: tuple(0 for _ in shape))


_mlp_call = pl.pallas_call(
    _mlp_body,
    grid=(NB,),
    in_specs=[
        pl.BlockSpec((NG, BLK, 128), lambda i: (0, i, 0)),
        pl.BlockSpec((NG, BLK, 128), lambda i: (0, i, 0)),
        _full((HID, HID // 2)),
        _full((1, HID // 2)),
        _full((HID // 2, HID // 4)),
        _full((1, HID // 4)),
        _full((HID // 4, D)),
        _full((1, D)),
        _full((HID, HID // 2)),
        _full((1, HID // 2)),
        _full((HID // 2, D)),
        _full((1, D)),
    ],
    out_specs=pl.BlockSpec((BLK,), lambda i: (i,)),
    out_shape=jax.ShapeDtypeStruct((B,), jnp.float32),
)


def kernel(user_feats, item_ids, item_feats_table, embed_table,
           uW1, ub1, uW2, ub2, uW3, ub3, iW1, ib1, iW2, ib2):
    up, ip = _sc_gather()(user_feats.T, item_ids, item_feats_table,
                          embed_table)
    scores = _mlp_call(up, ip,
                       uW1, ub1.reshape(1, -1), uW2, ub2.reshape(1, -1),
                       uW3, ub3.reshape(1, -1), iW1, ib1.reshape(1, -1),
                       iW2, ib2.reshape(1, -1))
    return scores
